# Initial kernel scaffold; baseline (speedup 1.0000x reference)
import jax
import jax.numpy as jnp
from jax.experimental import pallas as pl


def kernel(embedding, user2item_edge_index, reverse_edge_index, item2user_edge_index, social_edge_index, item2user_node_ids, u2i_Wl, u2i_Wr, u2i_bl, u2i_br, u2i_attn, rev_Wl, rev_Wr, rev_bl, rev_br, rev_attn, i2u_Wl, i2u_Wr, i2u_bl, i2u_br, i2u_attn, soc_Wl, soc_Wr, soc_bl, soc_br, soc_attn, out_W, out_b, bn_gamma, bn_beta):
    raise NotImplementedError("write your pallas kernel here")


# speedup vs baseline: 8.2170x; 8.2170x over previous
import functools

import jax
import jax.numpy as jnp
from jax import lax
from jax.experimental import pallas as pl
from jax.experimental.pallas import tpu as pltpu
from jax.experimental.pallas import tpu_sc as plsc

N = 10000
D = 128
E = 320000
NC = 2
NS = 16
NW = NC * NS
EW = E // NW
C = 80
NCHUNK = EW // C
RPT = 632
DN = 80
NA = N + DN

BR = 1000
G = N // BR

_mesh = functools.partial(
    plsc.VectorSubcoreMesh, core_axis_name="c", subcore_axis_name="s")


def _leaky(x, a):
  return jnp.maximum(x, a * x)


def _sc_edge_pass(src, dst, hs, hd, attn, zeros2d, ids=None):
  remap = ids is not None

  def body(*refs):
    it = iter(refs)
    src_hbm = next(it); dst_hbm = next(it)
    hs_hbm = next(it); hd_hbm = next(it)
    attn_hbm = next(it); z2_hbm = next(it)
    if remap:
      ids_hbm = next(it)
    p_hbm = next(it)
    src_v = next(it); dst_v = next(it)
    hs_rows = next(it); hd_rows = next(it)
    wbuf = next(it); denom_v = next(it); attn_v = next(it); didx = next(it)
    accum_sh = next(it)
    sem1 = next(it); sem2 = next(it)
    if remap:
      ids_v = next(it); src2_v = next(it); dst2_v = next(it)

    cid = lax.axis_index("c")
    sid = lax.axis_index("s")
    wid = sid * NC + cid

    @pl.when(sid < NS - 1)
    def _():
      pltpu.sync_copy(z2_hbm.at[pl.ds(sid * RPT, RPT)],
                      accum_sh.at[pl.ds(sid * RPT, RPT)])

    @pl.when(sid == NS - 1)
    def _():
      last = (NS - 1) * RPT
      pltpu.sync_copy(z2_hbm.at[pl.ds(last, NA - last)],
                      accum_sh.at[pl.ds(last, NA - last)])

    pltpu.sync_copy(z2_hbm.at[pl.ds(0, DN)], denom_v)
    pltpu.sync_copy(attn_hbm, attn_v)
    if remap:
      pltpu.sync_copy(ids_hbm, ids_v)
    iota16 = lax.iota(jnp.int32, 16)
    for m in range(DN // 16):
      didx[pl.ds(16 * m, 16)] = iota16 + (N + 16 * m)
    plsc.subcore_barrier()

    attn_regs = [attn_v[pl.ds(16 * k, 16)] for k in range(D // 16)]
    lane0 = iota16 == 0

    def chunk_body(g, carry):
      base = wid * EW + g * C
      pltpu.sync_copy(src_hbm.at[pl.ds(base, C)], src_v)
      pltpu.sync_copy(dst_hbm.at[pl.ds(base, C)], dst_v)
      if remap:
        for j in range(C // 16):
          sl = pl.ds(16 * j, 16)
          src2_v[sl] = plsc.load_gather(ids_v, [src_v[sl]])
          dst2_v[sl] = plsc.load_gather(ids_v, [dst_v[sl]])
        gsrc, gdst = src2_v, dst2_v
      else:
        gsrc, gdst = src_v, dst_v
      cp1 = pltpu.async_copy(hs_hbm.at[gsrc], hs_rows, sem1)
      cp2 = pltpu.async_copy(hd_hbm.at[gdst], hd_rows, sem2)
      cp1.wait()
      cp2.wait()

      def edge_body(e, acc0):
        acc = jnp.zeros((16,), jnp.float32)
        vals = []
        for k in range(D // 16):
          sl = pl.ds(16 * k, 16)
          a = hs_rows[e, sl]
          b = hd_rows[e, sl]
          s = a + b
          acc = acc + _leaky(s, 0.2) * attn_regs[k]
          vals.append(a)
        t = jnp.sum(acc)
        wv = jnp.exp(jnp.full((16,), t, jnp.float32))
        for k in range(D // 16):
          hs_rows[e, pl.ds(16 * k, 16)] = vals[k] * wv
        plsc.store_scatter(wbuf, [jnp.full((16,), e, jnp.int32)], wv,
                           mask=lane0)
        return acc0

      lax.fori_loop(0, C, edge_body, 0)

      for j in range(C // 16):
        sl = pl.ds(16 * j, 16)
        d16 = dst_v[sl]
        plsc.addupdate_scatter(
            denom_v, [jnp.right_shift(d16, 7), jnp.bitwise_and(d16, 127)],
            wbuf[sl])
      pltpu.sync_copy(hs_rows, accum_sh.at[dst_v], add=True)
      return carry

    lax.fori_loop(0, NCHUNK, chunk_body, 0)
    pltpu.sync_copy(denom_v, accum_sh.at[didx], add=True)
    plsc.subcore_barrier()

    @pl.when(sid < NS - 1)
    def _():
      pltpu.sync_copy(accum_sh.at[pl.ds(sid * RPT, RPT)],
                      p_hbm.at[cid, pl.ds(sid * RPT, RPT)])

    @pl.when(sid == NS - 1)
    def _():
      last = (NS - 1) * RPT
      pltpu.sync_copy(accum_sh.at[pl.ds(last, NA - last)],
                      p_hbm.at[cid, pl.ds(last, NA - last)])

  scratch = [
      pltpu.VMEM((C,), jnp.int32),
      pltpu.VMEM((C,), jnp.int32),
      pltpu.VMEM((C, D), jnp.float32),
      pltpu.VMEM((C, D), jnp.float32),
      pltpu.VMEM((C,), jnp.float32),
      pltpu.VMEM((DN, D), jnp.float32),
      pltpu.VMEM((D,), jnp.float32),
      pltpu.VMEM((DN,), jnp.int32),
      pltpu.VMEM_SHARED((NA, D), jnp.float32),
      pltpu.SemaphoreType.DMA,
      pltpu.SemaphoreType.DMA,
  ]
  if remap:
    scratch += [
        pltpu.VMEM((N,), jnp.int32),
        pltpu.VMEM((C,), jnp.int32),
        pltpu.VMEM((C,), jnp.int32),
    ]
  out_type = [jax.ShapeDtypeStruct((NC, NA, D), jnp.float32)]
  args = [src, dst, hs, hd, attn, zeros2d]
  if remap:
    args.append(ids)
  fn = pl.kernel(body, out_type=out_type, mesh=_mesh(),
                 compiler_params=pltpu.CompilerParams(
                     needs_layout_passes=False),
                 scratch_types=scratch)
  return fn(*args)[0]


def _sc_social_build(ids, stacked, rowsum):
  CH = 80
  NCH = N // CH
  ITER = (NCH + NW - 1) // NW

  def body(ids_hbm, st_hbm, rs_hbm, soc_hbm,
           ids_v, win_v, rs_v, winbuf, ridx_v, rows_v, win_sh, sem):
    cid = lax.axis_index("c")
    sid = lax.axis_index("s")
    wid = sid * NC + cid

    @pl.when(sid == 0)
    def _():
      pltpu.sync_copy(ids_hbm, ids_v)

      def zb(i, c):
        win_v[pl.ds(i * 16, 16)] = jnp.full((16,), -1, jnp.int32)
        return c
      lax.fori_loop(0, N // 16, zb, 0)

      lane0 = lax.iota(jnp.int32, 16) == 0

      def wb(gi, c):
        idv16 = ids_v[pl.ds(16 * gi, 16)]
        for l in range(16):
          plsc.store_scatter(win_v, [jnp.full((16,), idv16[l], jnp.int32)],
                             jnp.full((16,), 16 * gi + l, jnp.int32),
                             mask=lane0)
        return c
      lax.fori_loop(0, N // 16, wb, 0)
      pltpu.sync_copy(win_v, win_sh)

    plsc.subcore_barrier()
    pltpu.sync_copy(rs_hbm, rs_v)

    def cb(it, carry):
      g = wid + NW * it

      @pl.when(g < NCH)
      def _():
        base = g * CH
        pltpu.sync_copy(win_sh.at[pl.ds(base, CH)], winbuf)
        for j in range(CH // 16):
          sl = pl.ds(16 * j, 16)
          w16 = winbuf[sl]
          neg = w16 < 0
          winc = jnp.where(neg, 0, w16)
          rsg = plsc.load_gather(rs_v, [winc])
          sel = jnp.logical_and(jnp.logical_not(neg), rsg != 0.0)
          iota = lax.iota(jnp.int32, 16)
          ridx_v[sl] = jnp.where(sel, winc, base + 16 * j + N + iota)
        pltpu.async_copy(st_hbm.at[ridx_v], rows_v, sem).wait()
        pltpu.sync_copy(rows_v, soc_hbm.at[pl.ds(base, CH)])

      return carry

    lax.fori_loop(0, ITER, cb, 0)

  scratch = [
      pltpu.VMEM((N,), jnp.int32),
      pltpu.VMEM((N,), jnp.int32),
      pltpu.VMEM((N,), jnp.float32),
      pltpu.VMEM((CH,), jnp.int32),
      pltpu.VMEM((CH,), jnp.int32),
      pltpu.VMEM((CH, D), jnp.float32),
      pltpu.VMEM_SHARED((N,), jnp.int32),
      pltpu.SemaphoreType.DMA,
  ]
  fn = pl.kernel(body,
                 out_type=[jax.ShapeDtypeStruct((N, D), jnp.float32)],
                 mesh=_mesh(),
                 compiler_params=pltpu.CompilerParams(
                     needs_layout_passes=False),
                 scratch_types=scratch)
  return fn(ids, stacked, rowsum)


def _row_spec():
  return pl.BlockSpec((BR, D), lambda i: (i, 0))


def _full_spec(shape):
  nd = len(shape)
  return pl.BlockSpec(shape, lambda i: (0,) * nd)


def _tc_lin2(x, wl, bl, wr, br):
  def body(x_ref, wl_ref, bl_ref, wr_ref, br_ref, hs_ref, hd_ref):
    x = x_ref[...]
    hs_ref[...] = jnp.dot(x, wl_ref[...],
                          preferred_element_type=jnp.float32) + bl_ref[...]
    hd_ref[...] = jnp.dot(x, wr_ref[...],
                          preferred_element_type=jnp.float32) + br_ref[...]

  return pl.pallas_call(
      body, grid=(G,),
      in_specs=[_row_spec(), _full_spec((D, D)), _full_spec((1, D)),
                _full_spec((D, D)), _full_spec((1, D))],
      out_specs=[_row_spec(), _row_spec()],
      out_shape=[jax.ShapeDtypeStruct((N, D), jnp.float32)] * 2,
  )(x, wl, bl, wr, br)


def _fin(p_ref, d_ref):
  ps = p_ref[0] + p_ref[1]
  den = d_ref[0] + d_ref[1] + 1e-9
  return _leaky(ps / den, 0.01)


def _p_spec():
  return pl.BlockSpec((NC, BR, D), lambda i: (0, i, 0))


def _d_spec():
  return pl.BlockSpec((NC, BR, 1), lambda i: (0, i, 0))


def _tc_fin_lin2(p, d, wl, bl, wr, br):
  def body(p_ref, d_ref, wl_ref, bl_ref, wr_ref, br_ref, hs_ref, hd_ref):
    h = _fin(p_ref, d_ref)
    hs_ref[...] = jnp.dot(h, wl_ref[...],
                          preferred_element_type=jnp.float32) + bl_ref[...]
    hd_ref[...] = jnp.dot(h, wr_ref[...],
                          preferred_element_type=jnp.float32) + br_ref[...]

  return pl.pallas_call(
      body, grid=(G,),
      in_specs=[_p_spec(), _d_spec(), _full_spec((D, D)), _full_spec((1, D)),
                _full_spec((D, D)), _full_spec((1, D))],
      out_specs=[_row_spec(), _row_spec()],
      out_shape=[jax.ShapeDtypeStruct((N, D), jnp.float32)] * 2,
  )(p, d, wl, bl, wr, br)


def _tc_fin_rowsum(p, d):
  def body(p_ref, d_ref, h_ref, rs_ref):
    h = _fin(p_ref, d_ref)
    h_ref[...] = h
    rs_ref[...] = jnp.sum(h, axis=1, keepdims=True)

  return pl.pallas_call(
      body, grid=(G,),
      in_specs=[_p_spec(), _d_spec()],
      out_specs=[_row_spec(), pl.BlockSpec((BR, 1), lambda i: (i, 0))],
      out_shape=[jax.ShapeDtypeStruct((N, D), jnp.float32),
                 jax.ShapeDtypeStruct((N, 1), jnp.float32)],
  )(p, d)


def _tc_head(p2, d2, p3, d3, wt, wb, b):
  def body(p2_ref, d2_ref, p3_ref, d3_ref, wt_ref, wb_ref, b_ref,
           y_ref, st_ref):
    i = pl.program_id(0)
    h2 = _fin(p2_ref, d2_ref)
    s = _fin(p3_ref, d3_ref)
    y = (jnp.dot(h2, wt_ref[...], preferred_element_type=jnp.float32)
         + jnp.dot(s, wb_ref[...], preferred_element_type=jnp.float32)
         + b_ref[...])
    y_ref[...] = y

    @pl.when(i == 0)
    def _():
      st_ref[...] = jnp.zeros_like(st_ref)

    su = jnp.sum(y, axis=0)[None]
    sq = jnp.sum(y * y, axis=0)[None]
    pad = jnp.zeros((6, D), jnp.float32)
    st_ref[...] += jnp.concatenate([su, sq, pad], axis=0)

  return pl.pallas_call(
      body, grid=(G,),
      in_specs=[_p_spec(), _d_spec(), _p_spec(), _d_spec(),
                _full_spec((D, D)), _full_spec((D, D)), _full_spec((1, D))],
      out_specs=[_row_spec(), _full_spec((8, D))],
      out_shape=[jax.ShapeDtypeStruct((N, D), jnp.float32),
                 jax.ShapeDtypeStruct((8, D), jnp.float32)],
  )(p2, d2, p3, d3, wt, wb, b)


def _tc_norm(y, st, gamma, beta):
  def body(y_ref, st_ref, g_ref, b_ref, o_ref):
    mean = st_ref[0] / N
    var = st_ref[1] / N - mean * mean
    rstd = lax.rsqrt(var + 1e-5)
    o = (y_ref[...] - mean[None]) * (rstd[None] * g_ref[...]) + b_ref[...]
    o_ref[...] = _leaky(o, 0.01)

  return pl.pallas_call(
      body, grid=(G,),
      in_specs=[_row_spec(), _full_spec((8, D)), _full_spec((1, D)),
                _full_spec((1, D))],
      out_specs=_row_spec(),
      out_shape=jax.ShapeDtypeStruct((N, D), jnp.float32),
  )(y, st, gamma, beta)


def kernel(embedding, user2item_edge_index, reverse_edge_index,
           item2user_edge_index, social_edge_index, item2user_node_ids,
           u2i_Wl, u2i_Wr, u2i_bl, u2i_br, u2i_attn,
           rev_Wl, rev_Wr, rev_bl, rev_br, rev_attn,
           i2u_Wl, i2u_Wr, i2u_bl, i2u_br, i2u_attn,
           soc_Wl, soc_Wr, soc_bl, soc_br, soc_attn,
           out_W, out_b, bn_gamma, bn_beta):
  z2 = jnp.zeros((NA, D), jnp.float32)

  def r1(v):
    return v.reshape(1, D)

  def dn(pf):
    return pf[:, N:, :].reshape(NC, DN * D, 1)[:, :N]

  hs1, hd1 = _tc_lin2(embedding, u2i_Wl, r1(u2i_bl), u2i_Wr, r1(u2i_br))
  pf1 = _sc_edge_pass(user2item_edge_index[0], user2item_edge_index[1],
                      hs1, hd1, u2i_attn, z2)
  hs2, hd2 = _tc_fin_lin2(pf1, dn(pf1), rev_Wl, r1(rev_bl), rev_Wr, r1(rev_br))
  pf2 = _sc_edge_pass(reverse_edge_index[0], reverse_edge_index[1],
                      hs2, hd2, rev_attn, z2)

  A, B = _tc_lin2(embedding, i2u_Wl, r1(i2u_bl), i2u_Wr, r1(i2u_br))
  pfi = _sc_edge_pass(item2user_edge_index[0], item2user_edge_index[1],
                      A, B, i2u_attn, z2, ids=item2user_node_ids)
  i2u_emb, rowsum = _tc_fin_rowsum(pfi, dn(pfi))
  stacked = jnp.concatenate([i2u_emb, embedding], axis=0)
  social_emb = _sc_social_build(item2user_node_ids, stacked,
                                rowsum.reshape(N))
  (social_emb,) = social_emb if isinstance(social_emb, (list, tuple)) \
      else (social_emb,)

  hs3, hd3 = _tc_lin2(social_emb, soc_Wl, r1(soc_bl), soc_Wr, r1(soc_br))
  pf3 = _sc_edge_pass(social_edge_index[0], social_edge_index[1],
                      hs3, hd3, soc_attn, z2)

  y, st = _tc_head(pf2, dn(pf2), pf3, dn(pf3), out_W[:D], out_W[D:],
                   r1(out_b))
  return _tc_norm(y, st, r1(bn_gamma), r1(bn_beta))


# speedup vs baseline: 15.4481x; 1.8800x over previous
import functools

import jax
import jax.numpy as jnp
from jax import lax
from jax.experimental import pallas as pl
from jax.experimental.pallas import tpu as pltpu
from jax.experimental.pallas import tpu_sc as plsc

N = 10000
D = 128
E = 320000
NC = 2
NS = 16
NW = NC * NS
EW = E // NW
C = 80
NCHUNK = EW // C
RPT = 632
DN = 80
NA = N + DN

BR = 1000
G = N // BR

_mesh = functools.partial(
    plsc.VectorSubcoreMesh, core_axis_name="c", subcore_axis_name="s")


def _leaky(x, a):
  return jnp.maximum(x, a * x)


def _sc_edge_pass(src, dst, hs, hd, attn, zeros2d, ids=None):
  remap = ids is not None

  def body(*refs):
    it = iter(refs)
    src_hbm = next(it); dst_hbm = next(it)
    hs_hbm = next(it); hd_hbm = next(it)
    attn_hbm = next(it); z2_hbm = next(it)
    if remap:
      ids_hbm = next(it)
    p_hbm = next(it)
    src_v = next(it); dst_v = next(it)
    hs_rows = next(it); hd_rows = next(it)
    wbuf = next(it); denom_v = next(it); attn_v = next(it); didx = next(it)
    accum_sh = next(it)
    sem1 = next(it); sem2 = next(it)
    if remap:
      ids_v = next(it); src2_v = next(it); dst2_v = next(it)

    cid = lax.axis_index("c")
    sid = lax.axis_index("s")
    wid = sid * NC + cid

    @pl.when(sid < NS - 1)
    def _():
      pltpu.sync_copy(z2_hbm.at[pl.ds(sid * RPT, RPT)],
                      accum_sh.at[pl.ds(sid * RPT, RPT)])

    @pl.when(sid == NS - 1)
    def _():
      last = (NS - 1) * RPT
      pltpu.sync_copy(z2_hbm.at[pl.ds(last, NA - last)],
                      accum_sh.at[pl.ds(last, NA - last)])

    pltpu.sync_copy(z2_hbm.at[pl.ds(0, DN)], denom_v)
    pltpu.sync_copy(attn_hbm, attn_v)
    if remap:
      pltpu.sync_copy(ids_hbm, ids_v)
    iota16 = lax.iota(jnp.int32, 16)
    for m in range(DN // 16):
      didx[pl.ds(16 * m, 16)] = iota16 + (N + 16 * m)
    plsc.subcore_barrier()

    attn_regs = [attn_v[pl.ds(16 * k, 16)] for k in range(D // 16)]
    lane0 = iota16 == 0

    def chunk_body(g, carry):
      base = wid * EW + g * C
      pltpu.sync_copy(src_hbm.at[pl.ds(base, C)], src_v)
      pltpu.sync_copy(dst_hbm.at[pl.ds(base, C)], dst_v)
      if remap:
        for j in range(C // 16):
          sl = pl.ds(16 * j, 16)
          src2_v[sl] = plsc.load_gather(ids_v, [src_v[sl]])
          dst2_v[sl] = plsc.load_gather(ids_v, [dst_v[sl]])
        gsrc, gdst = src2_v, dst2_v
      else:
        gsrc, gdst = src_v, dst_v
      cp1 = pltpu.async_copy(hs_hbm.at[gsrc], hs_rows, sem1)
      cp2 = pltpu.async_copy(hd_hbm.at[gdst], hd_rows, sem2)
      cp1.wait()
      cp2.wait()

      @functools.partial(plsc.parallel_loop, 0, C, unroll=4)
      def _(e):
        acc = jnp.zeros((16,), jnp.float32)
        vals = []
        for k in range(D // 16):
          sl = pl.ds(16 * k, 16)
          a = hs_rows[e, sl]
          b = hd_rows[e, sl]
          s = a + b
          acc = acc + _leaky(s, 0.2) * attn_regs[k]
          vals.append(a)
        t = jnp.sum(acc)
        wv = jnp.exp(jnp.full((16,), t, jnp.float32))
        for k in range(D // 16):
          hs_rows[e, pl.ds(16 * k, 16)] = vals[k] * wv
        plsc.store_scatter(wbuf, [jnp.full((16,), e, jnp.int32)], wv,
                           mask=lane0)

      for j in range(C // 16):
        sl = pl.ds(16 * j, 16)
        d16 = dst_v[sl]
        plsc.addupdate_scatter(
            denom_v, [jnp.right_shift(d16, 7), jnp.bitwise_and(d16, 127)],
            wbuf[sl])
      pltpu.sync_copy(hs_rows, accum_sh.at[dst_v], add=True)
      return carry

    lax.fori_loop(0, NCHUNK, chunk_body, 0)
    pltpu.sync_copy(denom_v, accum_sh.at[didx], add=True)
    plsc.subcore_barrier()

    @pl.when(sid < NS - 1)
    def _():
      pltpu.sync_copy(accum_sh.at[pl.ds(sid * RPT, RPT)],
                      p_hbm.at[cid, pl.ds(sid * RPT, RPT)])

    @pl.when(sid == NS - 1)
    def _():
      last = (NS - 1) * RPT
      pltpu.sync_copy(accum_sh.at[pl.ds(last, NA - last)],
                      p_hbm.at[cid, pl.ds(last, NA - last)])

  scratch = [
      pltpu.VMEM((C,), jnp.int32),
      pltpu.VMEM((C,), jnp.int32),
      pltpu.VMEM((C, D), jnp.float32),
      pltpu.VMEM((C, D), jnp.float32),
      pltpu.VMEM((C,), jnp.float32),
      pltpu.VMEM((DN, D), jnp.float32),
      pltpu.VMEM((D,), jnp.float32),
      pltpu.VMEM((DN,), jnp.int32),
      pltpu.VMEM_SHARED((NA, D), jnp.float32),
      pltpu.SemaphoreType.DMA,
      pltpu.SemaphoreType.DMA,
  ]
  if remap:
    scratch += [
        pltpu.VMEM((N,), jnp.int32),
        pltpu.VMEM((C,), jnp.int32),
        pltpu.VMEM((C,), jnp.int32),
    ]
  out_type = [jax.ShapeDtypeStruct((NC, NA, D), jnp.float32)]
  args = [src, dst, hs, hd, attn, zeros2d]
  if remap:
    args.append(ids)
  fn = pl.kernel(body, out_type=out_type, mesh=_mesh(),
                 compiler_params=pltpu.CompilerParams(
                     needs_layout_passes=False),
                 scratch_types=scratch)
  return fn(*args)[0]


def _sc_social_build(ids, stacked, rowsum):
  CH = 80
  NCH = N // CH
  ITER = (NCH + NW - 1) // NW

  def body(ids_hbm, st_hbm, rs_hbm, soc_hbm,
           ids_v, win_v, rs_v, winbuf, ridx_v, rows_v, win_sh, sem):
    cid = lax.axis_index("c")
    sid = lax.axis_index("s")
    wid = sid * NC + cid

    @pl.when(sid == 0)
    def _():
      pltpu.sync_copy(ids_hbm, ids_v)

      def zb(i, c):
        win_v[pl.ds(i * 16, 16)] = jnp.full((16,), -1, jnp.int32)
        return c
      lax.fori_loop(0, N // 16, zb, 0)

      lane0 = lax.iota(jnp.int32, 16) == 0

      def wb(gi, c):
        idv16 = ids_v[pl.ds(16 * gi, 16)]
        for l in range(16):
          plsc.store_scatter(win_v, [jnp.full((16,), idv16[l], jnp.int32)],
                             jnp.full((16,), 16 * gi + l, jnp.int32),
                             mask=lane0)
        return c
      lax.fori_loop(0, N // 16, wb, 0)
      pltpu.sync_copy(win_v, win_sh)

    plsc.subcore_barrier()
    pltpu.sync_copy(rs_hbm, rs_v)

    def cb(it, carry):
      g = wid + NW * it

      @pl.when(g < NCH)
      def _():
        base = g * CH
        pltpu.sync_copy(win_sh.at[pl.ds(base, CH)], winbuf)
        for j in range(CH // 16):
          sl = pl.ds(16 * j, 16)
          w16 = winbuf[sl]
          neg = w16 < 0
          winc = jnp.where(neg, 0, w16)
          rsg = plsc.load_gather(rs_v, [winc])
          sel = jnp.logical_and(jnp.logical_not(neg), rsg != 0.0)
          iota = lax.iota(jnp.int32, 16)
          ridx_v[sl] = jnp.where(sel, winc, base + 16 * j + N + iota)
        pltpu.async_copy(st_hbm.at[ridx_v], rows_v, sem).wait()
        pltpu.sync_copy(rows_v, soc_hbm.at[pl.ds(base, CH)])

      return carry

    lax.fori_loop(0, ITER, cb, 0)

  scratch = [
      pltpu.VMEM((N,), jnp.int32),
      pltpu.VMEM((N,), jnp.int32),
      pltpu.VMEM((N,), jnp.float32),
      pltpu.VMEM((CH,), jnp.int32),
      pltpu.VMEM((CH,), jnp.int32),
      pltpu.VMEM((CH, D), jnp.float32),
      pltpu.VMEM_SHARED((N,), jnp.int32),
      pltpu.SemaphoreType.DMA,
  ]
  fn = pl.kernel(body,
                 out_type=[jax.ShapeDtypeStruct((N, D), jnp.float32)],
                 mesh=_mesh(),
                 compiler_params=pltpu.CompilerParams(
                     needs_layout_passes=False),
                 scratch_types=scratch)
  return fn(ids, stacked, rowsum)


def _row_spec():
  return pl.BlockSpec((BR, D), lambda i: (i, 0))


def _full_spec(shape):
  nd = len(shape)
  return pl.BlockSpec(shape, lambda i: (0,) * nd)


def _tc_lin2(x, wl, bl, wr, br):
  def body(x_ref, wl_ref, bl_ref, wr_ref, br_ref, hs_ref, hd_ref):
    x = x_ref[...]
    hs_ref[...] = jnp.dot(x, wl_ref[...],
                          preferred_element_type=jnp.float32) + bl_ref[...]
    hd_ref[...] = jnp.dot(x, wr_ref[...],
                          preferred_element_type=jnp.float32) + br_ref[...]

  return pl.pallas_call(
      body, grid=(G,),
      in_specs=[_row_spec(), _full_spec((D, D)), _full_spec((1, D)),
                _full_spec((D, D)), _full_spec((1, D))],
      out_specs=[_row_spec(), _row_spec()],
      out_shape=[jax.ShapeDtypeStruct((N, D), jnp.float32)] * 2,
  )(x, wl, bl, wr, br)


def _fin(p_ref, d_ref):
  ps = p_ref[0] + p_ref[1]
  den = d_ref[0] + d_ref[1] + 1e-9
  return _leaky(ps / den, 0.01)


def _p_spec():
  return pl.BlockSpec((NC, BR, D), lambda i: (0, i, 0))


def _d_spec():
  return pl.BlockSpec((NC, BR, 1), lambda i: (0, i, 0))


def _tc_fin_lin2(p, d, wl, bl, wr, br):
  def body(p_ref, d_ref, wl_ref, bl_ref, wr_ref, br_ref, hs_ref, hd_ref):
    h = _fin(p_ref, d_ref)
    hs_ref[...] = jnp.dot(h, wl_ref[...],
                          preferred_element_type=jnp.float32) + bl_ref[...]
    hd_ref[...] = jnp.dot(h, wr_ref[...],
                          preferred_element_type=jnp.float32) + br_ref[...]

  return pl.pallas_call(
      body, grid=(G,),
      in_specs=[_p_spec(), _d_spec(), _full_spec((D, D)), _full_spec((1, D)),
                _full_spec((D, D)), _full_spec((1, D))],
      out_specs=[_row_spec(), _row_spec()],
      out_shape=[jax.ShapeDtypeStruct((N, D), jnp.float32)] * 2,
  )(p, d, wl, bl, wr, br)


def _tc_fin_rowsum(p, d):
  def body(p_ref, d_ref, h_ref, rs_ref):
    h = _fin(p_ref, d_ref)
    h_ref[...] = h
    rs_ref[...] = jnp.sum(h, axis=1, keepdims=True)

  return pl.pallas_call(
      body, grid=(G,),
      in_specs=[_p_spec(), _d_spec()],
      out_specs=[_row_spec(), pl.BlockSpec((BR, 1), lambda i: (i, 0))],
      out_shape=[jax.ShapeDtypeStruct((N, D), jnp.float32),
                 jax.ShapeDtypeStruct((N, 1), jnp.float32)],
  )(p, d)


def _tc_head(p2, d2, p3, d3, wt, wb, b):
  def body(p2_ref, d2_ref, p3_ref, d3_ref, wt_ref, wb_ref, b_ref,
           y_ref, st_ref):
    i = pl.program_id(0)
    h2 = _fin(p2_ref, d2_ref)
    s = _fin(p3_ref, d3_ref)
    y = (jnp.dot(h2, wt_ref[...], preferred_element_type=jnp.float32)
         + jnp.dot(s, wb_ref[...], preferred_element_type=jnp.float32)
         + b_ref[...])
    y_ref[...] = y

    @pl.when(i == 0)
    def _():
      st_ref[...] = jnp.zeros_like(st_ref)

    su = jnp.sum(y, axis=0)[None]
    sq = jnp.sum(y * y, axis=0)[None]
    pad = jnp.zeros((6, D), jnp.float32)
    st_ref[...] += jnp.concatenate([su, sq, pad], axis=0)

  return pl.pallas_call(
      body, grid=(G,),
      in_specs=[_p_spec(), _d_spec(), _p_spec(), _d_spec(),
                _full_spec((D, D)), _full_spec((D, D)), _full_spec((1, D))],
      out_specs=[_row_spec(), _full_spec((8, D))],
      out_shape=[jax.ShapeDtypeStruct((N, D), jnp.float32),
                 jax.ShapeDtypeStruct((8, D), jnp.float32)],
  )(p2, d2, p3, d3, wt, wb, b)


def _tc_norm(y, st, gamma, beta):
  def body(y_ref, st_ref, g_ref, b_ref, o_ref):
    mean = st_ref[0] / N
    var = st_ref[1] / N - mean * mean
    rstd = lax.rsqrt(var + 1e-5)
    o = (y_ref[...] - mean[None]) * (rstd[None] * g_ref[...]) + b_ref[...]
    o_ref[...] = _leaky(o, 0.01)

  return pl.pallas_call(
      body, grid=(G,),
      in_specs=[_row_spec(), _full_spec((8, D)), _full_spec((1, D)),
                _full_spec((1, D))],
      out_specs=_row_spec(),
      out_shape=jax.ShapeDtypeStruct((N, D), jnp.float32),
  )(y, st, gamma, beta)


def kernel(embedding, user2item_edge_index, reverse_edge_index,
           item2user_edge_index, social_edge_index, item2user_node_ids,
           u2i_Wl, u2i_Wr, u2i_bl, u2i_br, u2i_attn,
           rev_Wl, rev_Wr, rev_bl, rev_br, rev_attn,
           i2u_Wl, i2u_Wr, i2u_bl, i2u_br, i2u_attn,
           soc_Wl, soc_Wr, soc_bl, soc_br, soc_attn,
           out_W, out_b, bn_gamma, bn_beta):
  z2 = jnp.zeros((NA, D), jnp.float32)

  def r1(v):
    return v.reshape(1, D)

  def dn(pf):
    return pf[:, N:, :].reshape(NC, DN * D, 1)[:, :N]

  hs1, hd1 = _tc_lin2(embedding, u2i_Wl, r1(u2i_bl), u2i_Wr, r1(u2i_br))
  pf1 = _sc_edge_pass(user2item_edge_index[0], user2item_edge_index[1],
                      hs1, hd1, u2i_attn, z2)
  hs2, hd2 = _tc_fin_lin2(pf1, dn(pf1), rev_Wl, r1(rev_bl), rev_Wr, r1(rev_br))
  pf2 = _sc_edge_pass(reverse_edge_index[0], reverse_edge_index[1],
                      hs2, hd2, rev_attn, z2)

  A, B = _tc_lin2(embedding, i2u_Wl, r1(i2u_bl), i2u_Wr, r1(i2u_br))
  pfi = _sc_edge_pass(item2user_edge_index[0], item2user_edge_index[1],
                      A, B, i2u_attn, z2, ids=item2user_node_ids)
  i2u_emb, rowsum = _tc_fin_rowsum(pfi, dn(pfi))
  stacked = jnp.concatenate([i2u_emb, embedding], axis=0)
  social_emb = _sc_social_build(item2user_node_ids, stacked,
                                rowsum.reshape(N))
  (social_emb,) = social_emb if isinstance(social_emb, (list, tuple)) \
      else (social_emb,)

  hs3, hd3 = _tc_lin2(social_emb, soc_Wl, r1(soc_bl), soc_Wr, r1(soc_br))
  pf3 = _sc_edge_pass(social_edge_index[0], social_edge_index[1],
                      hs3, hd3, soc_attn, z2)

  y, st = _tc_head(pf2, dn(pf2), pf3, dn(pf3), out_W[:D], out_W[D:],
                   r1(out_b))
  return _tc_norm(y, st, r1(bn_gamma), r1(bn_beta))
